# baseline (speedup 1.0000x reference)
import jax
import jax.numpy as jnp
from jax.experimental import pallas as pl
from jax.experimental.pallas import tpu as pltpu

_BT = 2048
_NBUF = 4


def _make_body(m, k, n):
    t_total = m // _BT

    def body(x_hbm, w_ref, b_ref, o_hbm, x_vmem, o_vmem, lsem, ssem):
        def load(t):
            b = t % _NBUF
            return pltpu.make_async_copy(
                x_hbm.at[pl.ds(t * _BT, _BT), :], x_vmem.at[b], lsem.at[b])

        def store(t):
            b = t % _NBUF
            return pltpu.make_async_copy(
                o_vmem.at[b], o_hbm.at[pl.ds(t * _BT, _BT), :], ssem.at[b])

        for t in range(min(_NBUF, t_total)):
            load(t).start()
        for t in range(t_total):
            b = t % _NBUF
            load(t).wait()
            if t >= _NBUF:
                store(t - _NBUF).wait()
            acc = jax.lax.dot_general(
                x_vmem[b], w_ref[:],
                dimension_numbers=(((1,), (1,)), ((), ())),
                preferred_element_type=jnp.float32,
            )
            o_vmem[b] = jnp.clip(acc + b_ref[:], 0.0, 1.0)
            if t + _NBUF < t_total:
                load(t + _NBUF).start()
            store(t).start()
        for t in range(max(0, t_total - _NBUF), t_total):
            store(t).wait()

    return body


def kernel(x, weight, bias):
    m, k = x.shape
    n = weight.shape[0]
    bias2d = bias.reshape(1, n)
    return pl.pallas_call(
        _make_body(m, k, n),
        in_specs=[
            pl.BlockSpec(memory_space=pl.ANY),
            pl.BlockSpec((n, k), lambda: (0, 0)),
            pl.BlockSpec((1, n), lambda: (0, 0)),
        ],
        out_specs=pl.BlockSpec(memory_space=pl.ANY),
        out_shape=jax.ShapeDtypeStruct((m, n), jnp.float32),
        scratch_shapes=[
            pltpu.VMEM((_NBUF, _BT, k), jnp.float32),
            pltpu.VMEM((_NBUF, _BT, n), jnp.float32),
            pltpu.SemaphoreType.DMA((_NBUF,)),
            pltpu.SemaphoreType.DMA((_NBUF,)),
        ],
    )(x, weight, bias2d)


# speedup vs baseline: 1.0197x; 1.0197x over previous
import jax
import jax.numpy as jnp
from jax.experimental import pallas as pl
from jax.experimental.pallas import tpu as pltpu

_BT = 512
_NBUF = 12


def _make_body(m, k, n):
    t_total = m // _BT

    def body(x_hbm, w_ref, b_ref, o_hbm, x_vmem, o_vmem, lsem, ssem):
        def load(t):
            b = t % _NBUF
            return pltpu.make_async_copy(
                x_hbm.at[pl.ds(t * _BT, _BT), :], x_vmem.at[b], lsem.at[b])

        def store(t):
            b = t % _NBUF
            return pltpu.make_async_copy(
                o_vmem.at[b], o_hbm.at[pl.ds(t * _BT, _BT), :], ssem.at[b])

        for t in range(min(_NBUF, t_total)):
            load(t).start()
        for t in range(t_total):
            b = t % _NBUF
            load(t).wait()
            if t >= _NBUF:
                store(t - _NBUF).wait()
            acc = jax.lax.dot_general(
                x_vmem[b], w_ref[:],
                dimension_numbers=(((1,), (1,)), ((), ())),
                preferred_element_type=jnp.float32,
            )
            o_vmem[b] = jnp.clip(acc + b_ref[:], 0.0, 1.0)
            if t + _NBUF < t_total:
                load(t + _NBUF).start()
            store(t).start()
        for t in range(max(0, t_total - _NBUF), t_total):
            store(t).wait()

    return body


def kernel(x, weight, bias):
    m, k = x.shape
    n = weight.shape[0]
    bias2d = bias.reshape(1, n)
    return pl.pallas_call(
        _make_body(m, k, n),
        in_specs=[
            pl.BlockSpec(memory_space=pl.ANY),
            pl.BlockSpec((n, k), lambda: (0, 0)),
            pl.BlockSpec((1, n), lambda: (0, 0)),
        ],
        out_specs=pl.BlockSpec(memory_space=pl.ANY),
        out_shape=jax.ShapeDtypeStruct((m, n), jnp.float32),
        scratch_shapes=[
            pltpu.VMEM((_NBUF, _BT, k), jnp.float32),
            pltpu.VMEM((_NBUF, _BT, n), jnp.float32),
            pltpu.SemaphoreType.DMA((_NBUF,)),
            pltpu.SemaphoreType.DMA((_NBUF,)),
        ],
    )(x, weight, bias2d)


# speedup vs baseline: 1.0305x; 1.0106x over previous
import jax
import jax.numpy as jnp
from jax.experimental import pallas as pl
from jax.experimental.pallas import tpu as pltpu

_BT = 1024
_NBUF = 6


def _make_body(m, k, n):
    t_total = m // _BT

    def body(x_hbm, w_ref, b_ref, o_hbm, x_vmem, o_vmem, lsem, ssem):
        def load(t):
            b = t % _NBUF
            return pltpu.make_async_copy(
                x_hbm.at[pl.ds(t * _BT, _BT), :], x_vmem.at[b], lsem.at[b])

        def store(t):
            b = t % _NBUF
            return pltpu.make_async_copy(
                o_vmem.at[b], o_hbm.at[pl.ds(t * _BT, _BT), :], ssem.at[b])

        for t in range(min(_NBUF, t_total)):
            load(t).start()
        for t in range(t_total):
            b = t % _NBUF
            load(t).wait()
            if t >= _NBUF:
                store(t - _NBUF).wait()
            acc = jax.lax.dot_general(
                x_vmem[b], w_ref[:],
                dimension_numbers=(((1,), (1,)), ((), ())),
                preferred_element_type=jnp.float32,
            )
            o_vmem[b] = jnp.clip(acc + b_ref[:], 0.0, 1.0)
            if t + _NBUF < t_total:
                load(t + _NBUF).start()
            store(t).start()
        for t in range(max(0, t_total - _NBUF), t_total):
            store(t).wait()

    return body


def kernel(x, weight, bias):
    m, k = x.shape
    n = weight.shape[0]
    bias2d = bias.reshape(1, n)
    return pl.pallas_call(
        _make_body(m, k, n),
        in_specs=[
            pl.BlockSpec(memory_space=pl.ANY),
            pl.BlockSpec((n, k), lambda: (0, 0)),
            pl.BlockSpec((1, n), lambda: (0, 0)),
        ],
        out_specs=pl.BlockSpec(memory_space=pl.ANY),
        out_shape=jax.ShapeDtypeStruct((m, n), jnp.float32),
        scratch_shapes=[
            pltpu.VMEM((_NBUF, _BT, k), jnp.float32),
            pltpu.VMEM((_NBUF, _BT, n), jnp.float32),
            pltpu.SemaphoreType.DMA((_NBUF,)),
            pltpu.SemaphoreType.DMA((_NBUF,)),
        ],
    )(x, weight, bias2d)


# speedup vs baseline: 1.0421x; 1.0112x over previous
import jax
import jax.numpy as jnp
from jax.experimental import pallas as pl
from jax.experimental.pallas import tpu as pltpu

_BT = 1024
_NBUF = 6
_SIZES = [512, 512] + [1024] * 14 + [512, 512]
_TILES = []
_off = 0
for _s in _SIZES:
    _TILES.append((_off, _s))
    _off += _s
assert _off == 16384


def _make_body(m, k, n):
    def body(x_hbm, w_ref, b_ref, o_hbm, x_vmem, o_vmem, lsem, ssem):
        def load(i):
            b = i % _NBUF
            off, rows = _TILES[i]
            return pltpu.make_async_copy(
                x_hbm.at[pl.ds(off, rows), :],
                x_vmem.at[b, pl.ds(0, rows), :], lsem.at[b])

        def store(i):
            b = i % _NBUF
            off, rows = _TILES[i]
            return pltpu.make_async_copy(
                o_vmem.at[b, pl.ds(0, rows), :],
                o_hbm.at[pl.ds(off, rows), :], ssem.at[b])

        t_total = len(_TILES)
        for i in range(min(_NBUF, t_total)):
            load(i).start()
        for i in range(t_total):
            b = i % _NBUF
            rows = _TILES[i][1]
            load(i).wait()
            if i >= _NBUF:
                store(i - _NBUF).wait()
            acc = jax.lax.dot_general(
                x_vmem[b, :rows], w_ref[:],
                dimension_numbers=(((1,), (1,)), ((), ())),
                preferred_element_type=jnp.float32,
            )
            o_vmem[b, :rows] = jnp.clip(acc + b_ref[:], 0.0, 1.0)
            if i + _NBUF < t_total:
                load(i + _NBUF).start()
            store(i).start()
        for i in range(max(0, t_total - _NBUF), t_total):
            store(i).wait()

    return body


def kernel(x, weight, bias):
    m, k = x.shape
    n = weight.shape[0]
    bias2d = bias.reshape(1, n)
    return pl.pallas_call(
        _make_body(m, k, n),
        in_specs=[
            pl.BlockSpec(memory_space=pl.ANY),
            pl.BlockSpec((n, k), lambda: (0, 0)),
            pl.BlockSpec((1, n), lambda: (0, 0)),
        ],
        out_specs=pl.BlockSpec(memory_space=pl.ANY),
        out_shape=jax.ShapeDtypeStruct((m, n), jnp.float32),
        scratch_shapes=[
            pltpu.VMEM((_NBUF, _BT, k), jnp.float32),
            pltpu.VMEM((_NBUF, _BT, n), jnp.float32),
            pltpu.SemaphoreType.DMA((_NBUF,)),
            pltpu.SemaphoreType.DMA((_NBUF,)),
        ],
    )(x, weight, bias2d)
